# baseline (speedup 1.0000x reference)
import functools

import jax
import jax.numpy as jnp
from jax import lax
from jax.experimental import pallas as pl
from jax.experimental.pallas import tpu as pltpu
from jax.experimental.pallas import tpu_sc as plsc

_T, _H, _E, _K = 2048, 1024, 8, 2
_F, _FS = 768, 2048
_M = 256
_NT = (_K * _T) // _M + _E
_NTM = _NT * _M
_MT = 256
_NW = 32
_CH = 32


def _router_body(x_ref, rw_ref, gp_ref, idx_ref, w_ref, gate_ref):
    x = x_ref[...]
    gate_ref[...] = lax.logistic(
        lax.dot_general(x, gp_ref[...], (((1,), (1,)), ((), ())),
                        preferred_element_type=jnp.float32))
    logits = lax.dot_general(x, rw_ref[...], (((1,), (1,)), ((), ())),
                             preferred_element_type=jnp.float32)
    lane = lax.broadcasted_iota(jnp.int32, (_T, _E), 1)
    m = jnp.max(logits, axis=1, keepdims=True)
    ex = jnp.exp(logits - m)
    probs = ex / jnp.sum(ex, axis=1, keepdims=True)
    big = jnp.int32(_E)
    m0 = jnp.max(probs, axis=1, keepdims=True)
    i0 = jnp.min(jnp.where(probs == m0, lane, big), axis=1, keepdims=True)
    probs1 = jnp.where(lane == i0, -1.0, probs)
    m1 = jnp.max(probs1, axis=1, keepdims=True)
    i1 = jnp.min(jnp.where(probs1 == m1, lane, big), axis=1, keepdims=True)
    s = m0 + m1
    idx_ref[...] = jnp.concatenate([i0, i1], axis=1)
    w_ref[...] = jnp.concatenate([m0 / s, m1 / s], axis=1)


def _expert_body(te_ref, x_ref, g_ref, u_ref, d_ref, dep_ref, o_ref,
                 gb_ref, ub_ref, db_ref):
    i = pl.program_id(0)
    act = te_ref[1, i] != 0
    changed = jnp.logical_or(
        i == 0, te_ref[0, i] != te_ref[0, jnp.maximum(i - 1, 0)])

    @pl.when(jnp.logical_and(act, changed))
    def _():
        gb_ref[...] = g_ref[0].astype(jnp.bfloat16).T
        ub_ref[...] = u_ref[0].astype(jnp.bfloat16).T
        db_ref[...] = d_ref[0].astype(jnp.bfloat16).T

    @pl.when(act)
    def _():
        xb = x_ref[...].astype(jnp.bfloat16)
        g = lax.dot_general(xb, gb_ref[...], (((1,), (0,)), ((), ())),
                            preferred_element_type=jnp.float32)
        u = lax.dot_general(xb, ub_ref[...], (((1,), (0,)), ((), ())),
                            preferred_element_type=jnp.float32)
        a = (g * lax.logistic(g) * u).astype(jnp.bfloat16)
        o_ref[...] = lax.dot_general(a, db_ref[...], (((1,), (0,)), ((), ())),
                                     preferred_element_type=jnp.float32)


def _shared_body(x_ref, sg_ref, su_ref, sd_ref, o_ref, sgb_ref, sub_ref,
                 sdb_ref):
    @pl.when(pl.program_id(0) == 0)
    def _():
        sgb_ref[...] = sg_ref[...].astype(jnp.bfloat16).T
        sub_ref[...] = su_ref[...].astype(jnp.bfloat16).T
        sdb_ref[...] = sd_ref[...].astype(jnp.bfloat16).T

    xb = x_ref[...].astype(jnp.bfloat16)
    g = lax.dot_general(xb, sgb_ref[...], (((1,), (0,)), ((), ())),
                        preferred_element_type=jnp.float32)
    u = lax.dot_general(xb, sub_ref[...], (((1,), (0,)), ((), ())),
                        preferred_element_type=jnp.float32)
    a = (g * lax.logistic(g) * u).astype(jnp.bfloat16)
    o_ref[...] = (lax.dot_general(a, sdb_ref[...], (((1,), (0,)), ((), ())),
                                  preferred_element_type=jnp.float32)
                  .astype(jnp.bfloat16))


def _combine_body(gate_ref, sa_ref, sb_ref, r0_ref, r1_ref, tw_ref, o_ref):
    tw = tw_ref[...]
    sh = (sa_ref[...].astype(jnp.float32) + sb_ref[...].astype(jnp.float32))
    o_ref[...] = (gate_ref[...] * sh
                  + tw[:, 0:1] * r0_ref[...]
                  + tw[:, 1:2] * r1_ref[...])


def _scatter_rows(hs, pos, nch, ch, n_out):
    idx3 = pos.reshape(_NW, nch, ch)
    mesh = plsc.VectorSubcoreMesh(core_axis_name="c", subcore_axis_name="s")

    @functools.partial(
        pl.kernel, mesh=mesh,
        out_type=jax.ShapeDtypeStruct((n_out, _H), jnp.float32),
        scratch_types=[
            pltpu.VMEM((nch, ch), jnp.int32),
            pltpu.VMEM((ch, _H), jnp.float32),
            pltpu.VMEM((ch, _H), jnp.float32),
            pltpu.SemaphoreType.DMA,
            pltpu.SemaphoreType.DMA,
        ],
    )
    def k(hs_hbm, idx_hbm, out_hbm, idx_v, b0, b1, s0, s1):
        wid = lax.axis_index("s") * 2 + lax.axis_index("c")
        row0 = (wid % 16) * (nch * ch)
        pltpu.sync_copy(idx_hbm.at[wid], idx_v)
        bufs, sems = [b0, b1], [s0, s1]
        wb = [None, None]
        for c in range(nch):
            b = c % 2
            if wb[b] is not None:
                wb[b].wait()
            pltpu.sync_copy(hs_hbm.at[pl.ds(row0 + c * ch, ch)], bufs[b])
            wb[b] = pltpu.async_copy(bufs[b], out_hbm.at[idx_v.at[c]], sems[b])
        wb[0].wait()
        if wb[1] is not None:
            wb[1].wait()

    return k(hs, idx3)


def _gather_rows(table, idx, nch, ch):
    d = table.shape[1]
    b = _NW * nch * ch
    idx3 = idx.reshape(_NW, nch, ch)
    mesh = plsc.VectorSubcoreMesh(core_axis_name="c", subcore_axis_name="s")

    @functools.partial(
        pl.kernel, mesh=mesh,
        out_type=jax.ShapeDtypeStruct((b, d), jnp.float32),
        scratch_types=[
            pltpu.VMEM((nch, ch), jnp.int32),
            pltpu.VMEM((ch, d), jnp.float32),
            pltpu.VMEM((ch, d), jnp.float32),
            pltpu.SemaphoreType.DMA,
            pltpu.SemaphoreType.DMA,
            pltpu.SemaphoreType.DMA,
            pltpu.SemaphoreType.DMA,
        ],
    )
    def k(table_hbm, idx_hbm, out_hbm, idx_v, r0, r1, g0, g1, w0, w1):
        wid = lax.axis_index("s") * 2 + lax.axis_index("c")
        base = wid * (nch * ch)
        pltpu.sync_copy(idx_hbm.at[wid], idx_v)
        bufs, gsem, wsem = [r0, r1], [g0, g1], [w0, w1]
        gat = [None] * nch
        wb = [None] * nch
        gat[0] = pltpu.async_copy(table_hbm.at[idx_v.at[0]], bufs[0], gsem[0])
        for c in range(nch):
            if c + 1 < nch:
                if c - 1 >= 0:
                    wb[c - 1].wait()
                gat[c + 1] = pltpu.async_copy(
                    table_hbm.at[idx_v.at[c + 1]], bufs[(c + 1) % 2],
                    gsem[(c + 1) % 2])
            gat[c].wait()
            wb[c] = pltpu.async_copy(
                bufs[c % 2], out_hbm.at[pl.ds(base + c * ch, ch)], wsem[c % 2])
        for c in range(max(0, nch - 2), nch):
            wb[c].wait()

    return k(table, idx3)


def kernel(hidden_states, router_w, gate_w, up_w, down_w, sg_w, su_w, sd_w,
           shared_gate_w):
    hs = hidden_states.reshape(_T, _H)

    top_i, top_w, gate = pl.pallas_call(
        _router_body,
        out_shape=(jax.ShapeDtypeStruct((_T, _K), jnp.int32),
                   jax.ShapeDtypeStruct((_T, _K), jnp.float32),
                   jax.ShapeDtypeStruct((_T, 1), jnp.float32)),
    )(hs, router_w, shared_gate_w)

    e_flat = jnp.concatenate([top_i[:, 0], top_i[:, 1]])
    onehot = (e_flat[:, None] == jnp.arange(_E)[None, :]).astype(jnp.int32)
    ranks_excl = jnp.cumsum(onehot, axis=0) - onehot
    rank = jnp.sum(ranks_excl * onehot, axis=1)
    counts = jnp.sum(onehot, axis=0)
    tiles_per_e = (counts + _M - 1) // _M
    tile_off = jnp.concatenate([jnp.zeros(1, jnp.int32),
                                jnp.cumsum(tiles_per_e)[:-1].astype(jnp.int32)])
    pos = jnp.sum(onehot * (tile_off * _M)[None, :], axis=1) + rank
    ends = tile_off + tiles_per_e
    te_raw = jnp.sum(jnp.arange(_NT)[:, None] >= ends[None, :], axis=1)
    last_e = jnp.max(jnp.where(counts > 0, jnp.arange(_E), -1))
    tile_expert = jnp.where(te_raw >= _E, last_e, te_raw).astype(jnp.int32)
    n_active = jnp.sum(tiles_per_e)
    active = (jnp.arange(_NT) < n_active).astype(jnp.int32)
    te_act = jnp.stack([tile_expert, active])

    ch1 = (2 * _T) // (_NW * 4)
    x_sorted = _scatter_rows(hs, pos, 4, ch1, _NTM)

    fs2 = _FS // 2

    def _shared_half(half):
        return pl.pallas_call(
            _shared_body,
            grid=(_T // _MT,),
            in_specs=[
                pl.BlockSpec((_MT, _H), lambda i: (i, 0)),
                pl.BlockSpec((fs2, _H), lambda i: (half, 0)),
                pl.BlockSpec((fs2, _H), lambda i: (half, 0)),
                pl.BlockSpec((_H, fs2), lambda i: (0, half)),
            ],
            out_specs=pl.BlockSpec((_MT, _H), lambda i: (i, 0)),
            out_shape=jax.ShapeDtypeStruct((_T, _H), jnp.bfloat16),
            scratch_shapes=[
                pltpu.VMEM((_H, fs2), jnp.bfloat16),
                pltpu.VMEM((_H, fs2), jnp.bfloat16),
                pltpu.VMEM((fs2, _H), jnp.bfloat16),
            ],
        )(hs, sg_w, su_w, sd_w)

    shared_a = _shared_half(0)

    grid_spec = pltpu.PrefetchScalarGridSpec(
        num_scalar_prefetch=1,
        grid=(_NT,),
        in_specs=[
            pl.BlockSpec((_M, _H), lambda i, te: (i, 0)),
            pl.BlockSpec((1, _F, _H), lambda i, te: (te[0, i], 0, 0)),
            pl.BlockSpec((1, _F, _H), lambda i, te: (te[0, i], 0, 0)),
            pl.BlockSpec((1, _H, _F), lambda i, te: (te[0, i], 0, 0)),
            pl.BlockSpec((8, 128), lambda i, te: (0, 0)),
        ],
        out_specs=pl.BlockSpec((_M, _H), lambda i, te: (i, 0)),
        scratch_shapes=[
            pltpu.VMEM((_H, _F), jnp.bfloat16),
            pltpu.VMEM((_H, _F), jnp.bfloat16),
            pltpu.VMEM((_F, _H), jnp.bfloat16),
        ],
    )
    contrib = pl.pallas_call(
        _expert_body,
        grid_spec=grid_spec,
        out_shape=jax.ShapeDtypeStruct((_NTM, _H), jnp.float32),
        compiler_params=pltpu.CompilerParams(
            dimension_semantics=("arbitrary",)),
    )(te_act, x_sorted, gate_w, up_w, down_w, shared_a)

    ch2 = (2 * _T) // (_NW * 4)
    rows_g = _gather_rows(contrib, pos, 4, ch2)

    shared_b = _shared_half(1)

    nmt = _T // _MT
    out = pl.pallas_call(
        _combine_body,
        grid=(nmt,),
        in_specs=[
            pl.BlockSpec((_MT, 1), lambda i: (i, 0)),
            pl.BlockSpec((_MT, _H), lambda i: (i, 0)),
            pl.BlockSpec((_MT, _H), lambda i: (i, 0)),
            pl.BlockSpec((_MT, _H), lambda i: (i, 0)),
            pl.BlockSpec((_MT, _H), lambda i: (i + nmt, 0)),
            pl.BlockSpec((_MT, _K), lambda i: (i, 0)),
        ],
        out_specs=pl.BlockSpec((_MT, _H), lambda i: (i, 0)),
        out_shape=jax.ShapeDtypeStruct((_T, _H), jnp.float32),
    )(gate, shared_a, shared_b, rows_g, rows_g, top_w)

    return out.reshape(1, _T, _H)


# speedup vs baseline: 1.0383x; 1.0383x over previous
import functools

import jax
import jax.numpy as jnp
from jax import lax
from jax.experimental import pallas as pl
from jax.experimental.pallas import tpu as pltpu
from jax.experimental.pallas import tpu_sc as plsc

_T, _H, _E, _K = 2048, 1024, 8, 2
_F, _FS = 768, 2048
_M = 256
_NT = (_K * _T) // _M + _E
_NTM = _NT * _M
_MT = 256
_NW = 32
_CH = 32


def _router_body(x_ref, rw_ref, gp_ref, idx_ref, w_ref, gate_ref):
    x = x_ref[...]
    gate_ref[...] = lax.logistic(
        lax.dot_general(x, gp_ref[...], (((1,), (1,)), ((), ())),
                        preferred_element_type=jnp.float32))
    logits = lax.dot_general(x, rw_ref[...], (((1,), (1,)), ((), ())),
                             preferred_element_type=jnp.float32)
    lane = lax.broadcasted_iota(jnp.int32, (_T, _E), 1)
    m = jnp.max(logits, axis=1, keepdims=True)
    ex = jnp.exp(logits - m)
    probs = ex / jnp.sum(ex, axis=1, keepdims=True)
    big = jnp.int32(_E)
    m0 = jnp.max(probs, axis=1, keepdims=True)
    i0 = jnp.min(jnp.where(probs == m0, lane, big), axis=1, keepdims=True)
    probs1 = jnp.where(lane == i0, -1.0, probs)
    m1 = jnp.max(probs1, axis=1, keepdims=True)
    i1 = jnp.min(jnp.where(probs1 == m1, lane, big), axis=1, keepdims=True)
    s = m0 + m1
    idx_ref[...] = jnp.concatenate([i0, i1], axis=1)
    w_ref[...] = jnp.concatenate([m0 / s, m1 / s], axis=1)


def _expert_body(te_ref, x_ref, g_ref, u_ref, d_ref, dep_ref, o_ref,
                 gb_ref, ub_ref, db_ref):
    i = pl.program_id(0)
    act = te_ref[1, i] != 0
    changed = jnp.logical_or(
        i == 0, te_ref[0, i] != te_ref[0, jnp.maximum(i - 1, 0)])

    @pl.when(jnp.logical_and(act, changed))
    def _():
        gb_ref[...] = g_ref[0].astype(jnp.bfloat16)
        ub_ref[...] = u_ref[0].astype(jnp.bfloat16)
        db_ref[...] = d_ref[0].astype(jnp.bfloat16)

    @pl.when(act)
    def _():
        xb = x_ref[...].astype(jnp.bfloat16)
        g = lax.dot_general(xb, gb_ref[...], (((1,), (1,)), ((), ())),
                            preferred_element_type=jnp.float32)
        u = lax.dot_general(xb, ub_ref[...], (((1,), (1,)), ((), ())),
                            preferred_element_type=jnp.float32)
        a = (g * lax.logistic(g) * u).astype(jnp.bfloat16)
        o_ref[...] = lax.dot_general(a, db_ref[...], (((1,), (1,)), ((), ())),
                                     preferred_element_type=jnp.float32)


def _shared_body(x_ref, sg_ref, su_ref, sd_ref, o_ref, sgb_ref, sub_ref,
                 sdb_ref):
    @pl.when(pl.program_id(0) == 0)
    def _():
        sgb_ref[...] = sg_ref[...].astype(jnp.bfloat16)
        sub_ref[...] = su_ref[...].astype(jnp.bfloat16)
        sdb_ref[...] = sd_ref[...].astype(jnp.bfloat16)

    xb = x_ref[...].astype(jnp.bfloat16)
    g = lax.dot_general(xb, sgb_ref[...], (((1,), (1,)), ((), ())),
                        preferred_element_type=jnp.float32)
    u = lax.dot_general(xb, sub_ref[...], (((1,), (1,)), ((), ())),
                        preferred_element_type=jnp.float32)
    a = (g * lax.logistic(g) * u).astype(jnp.bfloat16)
    o_ref[...] = (lax.dot_general(a, sdb_ref[...], (((1,), (1,)), ((), ())),
                                  preferred_element_type=jnp.float32)
                  .astype(jnp.bfloat16))


def _combine_body(gate_ref, sa_ref, sb_ref, r0_ref, r1_ref, tw_ref, o_ref):
    tw = tw_ref[...]
    sh = (sa_ref[...].astype(jnp.float32) + sb_ref[...].astype(jnp.float32))
    o_ref[...] = (gate_ref[...] * sh
                  + tw[:, 0:1] * r0_ref[...]
                  + tw[:, 1:2] * r1_ref[...])


def _scatter_rows(hs, pos, nch, ch, n_out):
    idx3 = pos.reshape(_NW, nch, ch)
    mesh = plsc.VectorSubcoreMesh(core_axis_name="c", subcore_axis_name="s")

    @functools.partial(
        pl.kernel, mesh=mesh,
        out_type=jax.ShapeDtypeStruct((n_out, _H), jnp.float32),
        scratch_types=[
            pltpu.VMEM((nch, ch), jnp.int32),
            pltpu.VMEM((ch, _H), jnp.float32),
            pltpu.VMEM((ch, _H), jnp.float32),
            pltpu.SemaphoreType.DMA,
            pltpu.SemaphoreType.DMA,
        ],
    )
    def k(hs_hbm, idx_hbm, out_hbm, idx_v, b0, b1, s0, s1):
        wid = lax.axis_index("s") * 2 + lax.axis_index("c")
        row0 = (wid % 16) * (nch * ch)
        pltpu.sync_copy(idx_hbm.at[wid], idx_v)
        bufs, sems = [b0, b1], [s0, s1]
        wb = [None, None]
        for c in range(nch):
            b = c % 2
            if wb[b] is not None:
                wb[b].wait()
            pltpu.sync_copy(hs_hbm.at[pl.ds(row0 + c * ch, ch)], bufs[b])
            wb[b] = pltpu.async_copy(bufs[b], out_hbm.at[idx_v.at[c]], sems[b])
        wb[0].wait()
        if wb[1] is not None:
            wb[1].wait()

    return k(hs, idx3)


def _gather_rows(table, idx, nch, ch):
    d = table.shape[1]
    b = _NW * nch * ch
    idx3 = idx.reshape(_NW, nch, ch)
    mesh = plsc.VectorSubcoreMesh(core_axis_name="c", subcore_axis_name="s")

    @functools.partial(
        pl.kernel, mesh=mesh,
        out_type=jax.ShapeDtypeStruct((b, d), jnp.float32),
        scratch_types=[
            pltpu.VMEM((nch, ch), jnp.int32),
            pltpu.VMEM((ch, d), jnp.float32),
            pltpu.VMEM((ch, d), jnp.float32),
            pltpu.SemaphoreType.DMA,
            pltpu.SemaphoreType.DMA,
            pltpu.SemaphoreType.DMA,
            pltpu.SemaphoreType.DMA,
        ],
    )
    def k(table_hbm, idx_hbm, out_hbm, idx_v, r0, r1, g0, g1, w0, w1):
        wid = lax.axis_index("s") * 2 + lax.axis_index("c")
        base = wid * (nch * ch)
        pltpu.sync_copy(idx_hbm.at[wid], idx_v)
        bufs, gsem, wsem = [r0, r1], [g0, g1], [w0, w1]
        gat = [None] * nch
        wb = [None] * nch
        gat[0] = pltpu.async_copy(table_hbm.at[idx_v.at[0]], bufs[0], gsem[0])
        for c in range(nch):
            if c + 1 < nch:
                if c - 1 >= 0:
                    wb[c - 1].wait()
                gat[c + 1] = pltpu.async_copy(
                    table_hbm.at[idx_v.at[c + 1]], bufs[(c + 1) % 2],
                    gsem[(c + 1) % 2])
            gat[c].wait()
            wb[c] = pltpu.async_copy(
                bufs[c % 2], out_hbm.at[pl.ds(base + c * ch, ch)], wsem[c % 2])
        for c in range(max(0, nch - 2), nch):
            wb[c].wait()

    return k(table, idx3)


def kernel(hidden_states, router_w, gate_w, up_w, down_w, sg_w, su_w, sd_w,
           shared_gate_w):
    hs = hidden_states.reshape(_T, _H)

    top_i, top_w, gate = pl.pallas_call(
        _router_body,
        out_shape=(jax.ShapeDtypeStruct((_T, _K), jnp.int32),
                   jax.ShapeDtypeStruct((_T, _K), jnp.float32),
                   jax.ShapeDtypeStruct((_T, 1), jnp.float32)),
    )(hs, router_w, shared_gate_w)

    e_flat = jnp.concatenate([top_i[:, 0], top_i[:, 1]])
    onehot = (e_flat[:, None] == jnp.arange(_E)[None, :]).astype(jnp.int32)
    ranks_excl = jnp.cumsum(onehot, axis=0) - onehot
    rank = jnp.sum(ranks_excl * onehot, axis=1)
    counts = jnp.sum(onehot, axis=0)
    tiles_per_e = (counts + _M - 1) // _M
    tile_off = jnp.concatenate([jnp.zeros(1, jnp.int32),
                                jnp.cumsum(tiles_per_e)[:-1].astype(jnp.int32)])
    pos = jnp.sum(onehot * (tile_off * _M)[None, :], axis=1) + rank
    ends = tile_off + tiles_per_e
    te_raw = jnp.sum(jnp.arange(_NT)[:, None] >= ends[None, :], axis=1)
    last_e = jnp.max(jnp.where(counts > 0, jnp.arange(_E), -1))
    tile_expert = jnp.where(te_raw >= _E, last_e, te_raw).astype(jnp.int32)
    n_active = jnp.sum(tiles_per_e)
    active = (jnp.arange(_NT) < n_active).astype(jnp.int32)
    te_act = jnp.stack([tile_expert, active])

    ch1 = (2 * _T) // (_NW * 4)
    x_sorted = _scatter_rows(hs, pos, 4, ch1, _NTM)

    fs2 = _FS // 2

    def _shared_half(half):
        return pl.pallas_call(
            _shared_body,
            grid=(_T // _MT,),
            in_specs=[
                pl.BlockSpec((_MT, _H), lambda i: (i, 0)),
                pl.BlockSpec((fs2, _H), lambda i: (half, 0)),
                pl.BlockSpec((fs2, _H), lambda i: (half, 0)),
                pl.BlockSpec((_H, fs2), lambda i: (0, half)),
            ],
            out_specs=pl.BlockSpec((_MT, _H), lambda i: (i, 0)),
            out_shape=jax.ShapeDtypeStruct((_T, _H), jnp.bfloat16),
            scratch_shapes=[
                pltpu.VMEM((fs2, _H), jnp.bfloat16),
                pltpu.VMEM((fs2, _H), jnp.bfloat16),
                pltpu.VMEM((_H, fs2), jnp.bfloat16),
            ],
        )(hs, sg_w, su_w, sd_w)

    shared_a = _shared_half(0)

    grid_spec = pltpu.PrefetchScalarGridSpec(
        num_scalar_prefetch=1,
        grid=(_NT,),
        in_specs=[
            pl.BlockSpec((_M, _H), lambda i, te: (i, 0)),
            pl.BlockSpec((1, _F, _H), lambda i, te: (te[0, i], 0, 0)),
            pl.BlockSpec((1, _F, _H), lambda i, te: (te[0, i], 0, 0)),
            pl.BlockSpec((1, _H, _F), lambda i, te: (te[0, i], 0, 0)),
            pl.BlockSpec((8, 128), lambda i, te: (0, 0)),
        ],
        out_specs=pl.BlockSpec((_M, _H), lambda i, te: (i, 0)),
        scratch_shapes=[
            pltpu.VMEM((_F, _H), jnp.bfloat16),
            pltpu.VMEM((_F, _H), jnp.bfloat16),
            pltpu.VMEM((_H, _F), jnp.bfloat16),
        ],
    )
    contrib = pl.pallas_call(
        _expert_body,
        grid_spec=grid_spec,
        out_shape=jax.ShapeDtypeStruct((_NTM, _H), jnp.float32),
        compiler_params=pltpu.CompilerParams(
            dimension_semantics=("arbitrary",)),
    )(te_act, x_sorted, gate_w, up_w, down_w, shared_a)

    ch2 = (2 * _T) // (_NW * 4)
    rows_g = _gather_rows(contrib, pos, 4, ch2)

    shared_b = _shared_half(1)

    nmt = _T // _MT
    out = pl.pallas_call(
        _combine_body,
        grid=(nmt,),
        in_specs=[
            pl.BlockSpec((_MT, 1), lambda i: (i, 0)),
            pl.BlockSpec((_MT, _H), lambda i: (i, 0)),
            pl.BlockSpec((_MT, _H), lambda i: (i, 0)),
            pl.BlockSpec((_MT, _H), lambda i: (i, 0)),
            pl.BlockSpec((_MT, _H), lambda i: (i + nmt, 0)),
            pl.BlockSpec((_MT, _K), lambda i: (i, 0)),
        ],
        out_specs=pl.BlockSpec((_MT, _H), lambda i: (i, 0)),
        out_shape=jax.ShapeDtypeStruct((_T, _H), jnp.float32),
    )(gate, shared_a, shared_b, rows_g, rows_g, top_w)

    return out.reshape(1, _T, _H)


# speedup vs baseline: 1.0393x; 1.0009x over previous
import functools

import jax
import jax.numpy as jnp
from jax import lax
from jax.experimental import pallas as pl
from jax.experimental.pallas import tpu as pltpu
from jax.experimental.pallas import tpu_sc as plsc

_T, _H, _E, _K = 2048, 1024, 8, 2
_F, _FS = 768, 2048
_M = 256
_NT = (_K * _T) // _M + _E
_NTM = _NT * _M
_MT = 256
_NW = 32


def _router_body(x_ref, rw_ref, gp_ref, idx_ref, w_ref, gate_ref):
    x = x_ref[...]
    gate_ref[...] = lax.logistic(
        lax.dot_general(x, gp_ref[...], (((1,), (1,)), ((), ())),
                        preferred_element_type=jnp.float32))
    logits = lax.dot_general(x, rw_ref[...], (((1,), (1,)), ((), ())),
                             preferred_element_type=jnp.float32)
    lane = lax.broadcasted_iota(jnp.int32, (_T, _E), 1)
    m = jnp.max(logits, axis=1, keepdims=True)
    ex = jnp.exp(logits - m)
    probs = ex / jnp.sum(ex, axis=1, keepdims=True)
    big = jnp.int32(_E)
    m0 = jnp.max(probs, axis=1, keepdims=True)
    i0 = jnp.min(jnp.where(probs == m0, lane, big), axis=1, keepdims=True)
    probs1 = jnp.where(lane == i0, -1.0, probs)
    m1 = jnp.max(probs1, axis=1, keepdims=True)
    i1 = jnp.min(jnp.where(probs1 == m1, lane, big), axis=1, keepdims=True)
    s = m0 + m1
    idx_ref[...] = jnp.concatenate([i0, i1], axis=1)
    w_ref[...] = jnp.concatenate([m0 / s, m1 / s], axis=1)


def _expert_body(te_ref, x_ref, g_ref, u_ref, d_ref, dep_ref, o_ref,
                 gb_ref, ub_ref, db_ref):
    i = pl.program_id(0)
    act = te_ref[1, i] != 0
    changed = jnp.logical_or(
        i == 0, te_ref[0, i] != te_ref[0, jnp.maximum(i - 1, 0)])

    @pl.when(jnp.logical_and(act, changed))
    def _():
        gb_ref[...] = g_ref[0].astype(jnp.bfloat16)
        ub_ref[...] = u_ref[0].astype(jnp.bfloat16)
        db_ref[...] = d_ref[0].astype(jnp.bfloat16)

    @pl.when(act)
    def _():
        xb = x_ref[...].astype(jnp.bfloat16)
        g = lax.dot_general(xb, gb_ref[...], (((1,), (1,)), ((), ())),
                            preferred_element_type=jnp.float32)
        u = lax.dot_general(xb, ub_ref[...], (((1,), (1,)), ((), ())),
                            preferred_element_type=jnp.float32)
        a = (g * lax.logistic(g) * u).astype(jnp.bfloat16)
        o_ref[...] = lax.dot_general(a, db_ref[...], (((1,), (1,)), ((), ())),
                                     preferred_element_type=jnp.float32)


def _shared_body(x_ref, sg_ref, su_ref, sd_ref, o_ref, sgb_ref, sub_ref,
                 sdb_ref):
    @pl.when(pl.program_id(0) == 0)
    def _():
        sgb_ref[...] = sg_ref[...].astype(jnp.bfloat16)
        sub_ref[...] = su_ref[...].astype(jnp.bfloat16)
        sdb_ref[...] = sd_ref[...].astype(jnp.bfloat16)

    xb = x_ref[...].astype(jnp.bfloat16)
    g = lax.dot_general(xb, sgb_ref[...], (((1,), (1,)), ((), ())),
                        preferred_element_type=jnp.float32)
    u = lax.dot_general(xb, sub_ref[...], (((1,), (1,)), ((), ())),
                        preferred_element_type=jnp.float32)
    a = (g * lax.logistic(g) * u).astype(jnp.bfloat16)
    o_ref[...] = (lax.dot_general(a, sdb_ref[...], (((1,), (1,)), ((), ())),
                                  preferred_element_type=jnp.float32)
                  .astype(jnp.bfloat16))


def _combine_body(gate_ref, sa_ref, sb_ref, r0_ref, r1_ref, tw_ref, o_ref):
    tw = tw_ref[...]
    sh = (sa_ref[...].astype(jnp.float32) + sb_ref[...].astype(jnp.float32))
    o_ref[...] = (gate_ref[...] * sh
                  + tw[:, 0:1] * r0_ref[...]
                  + tw[:, 1:2] * r1_ref[...])


def _scatter_rows(hs, pos, nch, ch, n_out):
    idx3 = pos.reshape(_NW, nch, ch)
    mesh = plsc.VectorSubcoreMesh(core_axis_name="c", subcore_axis_name="s")

    @functools.partial(
        pl.kernel, mesh=mesh,
        out_type=jax.ShapeDtypeStruct((n_out, _H), jnp.float32),
        scratch_types=[
            pltpu.VMEM((nch, ch), jnp.int32),
            pltpu.VMEM((ch, _H), jnp.float32),
            pltpu.VMEM((ch, _H), jnp.float32),
            pltpu.SemaphoreType.DMA,
            pltpu.SemaphoreType.DMA,
        ],
    )
    def k(hs_hbm, idx_hbm, out_hbm, idx_v, b0, b1, s0, s1):
        wid = lax.axis_index("s") * 2 + lax.axis_index("c")
        row0 = (wid % 16) * (nch * ch)
        pltpu.sync_copy(idx_hbm.at[wid], idx_v)
        bufs, sems = [b0, b1], [s0, s1]
        wb = [None, None]
        for c in range(nch):
            b = c % 2
            if wb[b] is not None:
                wb[b].wait()
            pltpu.sync_copy(hs_hbm.at[pl.ds(row0 + c * ch, ch)], bufs[b])
            wb[b] = pltpu.async_copy(bufs[b], out_hbm.at[idx_v.at[c]], sems[b])
        wb[0].wait()
        if wb[1] is not None:
            wb[1].wait()

    return k(hs, idx3)


def _gather_rows(table, idx, nch, ch):
    d = table.shape[1]
    b = _NW * nch * ch
    idx3 = idx.reshape(_NW, nch, ch)
    mesh = plsc.VectorSubcoreMesh(core_axis_name="c", subcore_axis_name="s")

    @functools.partial(
        pl.kernel, mesh=mesh,
        out_type=jax.ShapeDtypeStruct((b, d), jnp.float32),
        scratch_types=[
            pltpu.VMEM((nch, ch), jnp.int32),
            pltpu.VMEM((ch, d), jnp.float32),
            pltpu.VMEM((ch, d), jnp.float32),
            pltpu.SemaphoreType.DMA,
            pltpu.SemaphoreType.DMA,
            pltpu.SemaphoreType.DMA,
            pltpu.SemaphoreType.DMA,
        ],
    )
    def k(table_hbm, idx_hbm, out_hbm, idx_v, r0, r1, g0, g1, w0, w1):
        wid = lax.axis_index("s") * 2 + lax.axis_index("c")
        base = wid * (nch * ch)
        pltpu.sync_copy(idx_hbm.at[wid], idx_v)
        bufs, gsem, wsem = [r0, r1], [g0, g1], [w0, w1]
        gat = [None] * nch
        wb = [None] * nch
        gat[0] = pltpu.async_copy(table_hbm.at[idx_v.at[0]], bufs[0], gsem[0])
        for c in range(nch):
            if c + 1 < nch:
                if c - 1 >= 0:
                    wb[c - 1].wait()
                gat[c + 1] = pltpu.async_copy(
                    table_hbm.at[idx_v.at[c + 1]], bufs[(c + 1) % 2],
                    gsem[(c + 1) % 2])
            gat[c].wait()
            wb[c] = pltpu.async_copy(
                bufs[c % 2], out_hbm.at[pl.ds(base + c * ch, ch)], wsem[c % 2])
        for c in range(max(0, nch - 2), nch):
            wb[c].wait()

    return k(table, idx3)


def kernel(hidden_states, router_w, gate_w, up_w, down_w, sg_w, su_w, sd_w,
           shared_gate_w):
    hs = hidden_states.reshape(_T, _H)

    top_i, top_w, gate = pl.pallas_call(
        _router_body,
        out_shape=(jax.ShapeDtypeStruct((_T, _K), jnp.int32),
                   jax.ShapeDtypeStruct((_T, _K), jnp.float32),
                   jax.ShapeDtypeStruct((_T, 1), jnp.float32)),
    )(hs, router_w, shared_gate_w)

    e_flat = jnp.concatenate([top_i[:, 0], top_i[:, 1]])
    onehot = (e_flat[:, None] == jnp.arange(_E)[None, :]).astype(jnp.int32)
    ranks_excl = jnp.cumsum(onehot, axis=0) - onehot
    rank = jnp.sum(ranks_excl * onehot, axis=1)
    counts = jnp.sum(onehot, axis=0)
    tiles_per_e = (counts + _M - 1) // _M
    tile_off = jnp.concatenate([jnp.zeros(1, jnp.int32),
                                jnp.cumsum(tiles_per_e)[:-1].astype(jnp.int32)])
    pos = jnp.sum(onehot * (tile_off * _M)[None, :], axis=1) + rank
    ends = tile_off + tiles_per_e
    te_raw = jnp.sum(jnp.arange(_NT)[:, None] >= ends[None, :], axis=1)
    last_e = jnp.max(jnp.where(counts > 0, jnp.arange(_E), -1))
    tile_expert = jnp.where(te_raw >= _E, last_e, te_raw).astype(jnp.int32)
    n_active = jnp.sum(tiles_per_e)
    active = (jnp.arange(_NT) < n_active).astype(jnp.int32)
    te_act = jnp.stack([tile_expert, active])

    ch1 = (2 * _T) // (_NW * 4)
    x_sorted = _scatter_rows(hs, pos, 4, ch1, _NTM)

    fs2 = _FS // 2

    def _shared_half(half):
        return pl.pallas_call(
            _shared_body,
            grid=(_T // _MT,),
            in_specs=[
                pl.BlockSpec((_MT, _H), lambda i: (i, 0)),
                pl.BlockSpec((fs2, _H), lambda i: (half, 0)),
                pl.BlockSpec((fs2, _H), lambda i: (half, 0)),
                pl.BlockSpec((_H, fs2), lambda i: (0, half)),
            ],
            out_specs=pl.BlockSpec((_MT, _H), lambda i: (i, 0)),
            out_shape=jax.ShapeDtypeStruct((_T, _H), jnp.bfloat16),
            scratch_shapes=[
                pltpu.VMEM((fs2, _H), jnp.bfloat16),
                pltpu.VMEM((fs2, _H), jnp.bfloat16),
                pltpu.VMEM((_H, fs2), jnp.bfloat16),
            ],
        )(hs, sg_w, su_w, sd_w)

    shared_a = _shared_half(0)

    grid_spec = pltpu.PrefetchScalarGridSpec(
        num_scalar_prefetch=1,
        grid=(_NT,),
        in_specs=[
            pl.BlockSpec((_M, _H), lambda i, te: (i, 0)),
            pl.BlockSpec((1, _F, _H), lambda i, te: (te[0, i], 0, 0)),
            pl.BlockSpec((1, _F, _H), lambda i, te: (te[0, i], 0, 0)),
            pl.BlockSpec((1, _H, _F), lambda i, te: (te[0, i], 0, 0)),
            pl.BlockSpec((8, 128), lambda i, te: (0, 0)),
        ],
        out_specs=pl.BlockSpec((_M, _H), lambda i, te: (i, 0)),
        scratch_shapes=[
            pltpu.VMEM((_F, _H), jnp.bfloat16),
            pltpu.VMEM((_F, _H), jnp.bfloat16),
            pltpu.VMEM((_H, _F), jnp.bfloat16),
        ],
    )
    contrib = pl.pallas_call(
        _expert_body,
        grid_spec=grid_spec,
        out_shape=jax.ShapeDtypeStruct((_NTM, _H), jnp.float32),
        compiler_params=pltpu.CompilerParams(
            dimension_semantics=("arbitrary",)),
    )(te_act, x_sorted, gate_w, up_w, down_w, shared_a)

    ch2 = (2 * _T) // (_NW * 4)
    rows_g = _gather_rows(contrib, pos, 4, ch2)

    shared_b = _shared_half(1)

    nmt = _T // _MT
    out = pl.pallas_call(
        _combine_body,
        grid=(nmt,),
        in_specs=[
            pl.BlockSpec((_MT, 1), lambda i: (i, 0)),
            pl.BlockSpec((_MT, _H), lambda i: (i, 0)),
            pl.BlockSpec((_MT, _H), lambda i: (i, 0)),
            pl.BlockSpec((_MT, _H), lambda i: (i, 0)),
            pl.BlockSpec((_MT, _H), lambda i: (i + nmt, 0)),
            pl.BlockSpec((_MT, _K), lambda i: (i, 0)),
        ],
        out_specs=pl.BlockSpec((_MT, _H), lambda i: (i, 0)),
        out_shape=jax.ShapeDtypeStruct((_T, _H), jnp.float32),
    )(gate, shared_a, shared_b, rows_g, rows_g, top_w)

    return out.reshape(1, _T, _H)
